# baseline (speedup 1.0000x reference)
import functools

import jax
import jax.numpy as jnp
from jax import lax
from jax.experimental import pallas as pl
from jax.experimental.pallas import tpu as pltpu
from jax.experimental.pallas import tpu_sc as plsc

NUM_WORDS = 1000000
NUM_HASH = 3
WIDTH = 32

NC = 2
NS = 16
NW = NC * NS
LANES = 16
ROWS = 8
T = ROWS * 50
NBUF = 2
PIECES = [(0, T)]


def _body(x_hbm, ht_hbm, imp_hbm, emb_hbm, out_hbm,
          x_v, iidx_v, hrows_v, irows_v, emb_v, out_v,
          sem_h, sem_i, sem_e):
    n_chunks = x_v.shape[0]
    G = T // LANES
    wid = lax.axis_index("s") * NC + lax.axis_index("c")

    pltpu.sync_copy(x_hbm.at[wid], x_v)

    def idx_body(i, _):
        c = i // G
        sl = pl.ds((i % G) * LANES, LANES)
        xi = x_v[c, sl] + 3
        iidx_v[c, sl] = jnp.where(xi >= NUM_WORDS, xi - NUM_WORDS, xi)
        return _
    lax.fori_loop(0, n_chunks * G, idx_body, None)

    def issue_h(c, b):
        for j in range(NUM_HASH):
            seg = ht_hbm.at[pl.ds(j * NUM_WORDS, NUM_WORDS)]
            for o, s in PIECES:
                pltpu.async_copy(seg.at[x_v.at[c, pl.ds(o, s)]],
                                 hrows_v.at[b, j, pl.ds(o, s)], sem_h[b])

    def issue_i(c, b):
        for j in range(NUM_HASH):
            seg = imp_hbm.at[pl.ds(j * NUM_WORDS, NUM_WORDS)]
            for o, s in PIECES:
                pltpu.async_copy(seg.at[iidx_v.at[c, pl.ds(o, s)]],
                                 irows_v.at[b, j, pl.ds(o, s)], sem_i[b])

    def issue_hi(c, b):
        issue_h(c, b)
        issue_i(c, b)

    def wait_h(c, b):
        for j in range(NUM_HASH):
            seg = ht_hbm.at[pl.ds(j * NUM_WORDS, NUM_WORDS)]
            for o, s in PIECES:
                pltpu.make_async_copy(seg.at[x_v.at[c, pl.ds(o, s)]],
                                      hrows_v.at[b, j, pl.ds(o, s)],
                                      sem_h[b]).wait()

    def wait_i(c, b):
        for j in range(NUM_HASH):
            seg = imp_hbm.at[pl.ds(j * NUM_WORDS, NUM_WORDS)]
            for o, s in PIECES:
                pltpu.make_async_copy(seg.at[iidx_v.at[c, pl.ds(o, s)]],
                                      irows_v.at[b, j, pl.ds(o, s)],
                                      sem_i[b]).wait()

    def issue_e(b):
        for j in range(NUM_HASH):
            pltpu.async_copy(emb_hbm.at[hrows_v.at[b, j]], emb_v.at[b, j],
                             sem_e[b])

    def wait_e(b):
        for j in range(NUM_HASH):
            pltpu.make_async_copy(emb_hbm.at[hrows_v.at[b, j]],
                                  emb_v.at[b, j], sem_e[b]).wait()

    def compute(c, b):
        def grp_body(g, _):
            sl = pl.ds(g * LANES, LANES)
            vi = [irows_v[b, j, sl] for j in range(NUM_HASH)]
            for k in range(LANES):
                t = g * LANES + k
                r = t // 50
                l = t - r * 50
                w0, w1, w2 = vi[0][k], vi[1][k], vi[2][k]
                for h in range(0, WIDTH, LANES):
                    acc = (w0 * emb_v[b, 0, t, pl.ds(h, LANES)]
                           + w1 * emb_v[b, 1, t, pl.ds(h, LANES)]
                           + w2 * emb_v[b, 2, t, pl.ds(h, LANES)])
                    out_v[r, l, pl.ds(h, LANES)] = acc
            return _
        lax.fori_loop(0, G, grp_body, None)
        pltpu.sync_copy(out_v,
                        out_hbm.at[pl.ds(wid * (n_chunks * ROWS) + c * ROWS,
                                         ROWS)])

    issue_hi(0, 0)
    issue_hi(1, 1)
    wait_h(0, 0)
    issue_e(0)

    def step(c, b):
        b2 = (b + 1) % NBUF

        @pl.when(c + 1 < n_chunks)
        def _():
            wait_h(c + 1, b2)
            issue_e(b2)
        wait_e(b)
        wait_i(c, b)

        @pl.when(c + 2 < n_chunks)
        def _():
            issue_h(c + 2, b)
        compute(c, b)

        @pl.when(c + 2 < n_chunks)
        def _():
            issue_i(c + 2, b)

    def loop_body(i, _):
        for b in range(NBUF):
            step(i * NBUF + b, b)
        return _
    lax.fori_loop(0, n_chunks // NBUF, loop_body, None)


def kernel(X, hash_table, word_importance, embedding):
    B, L = X.shape
    N = B * L
    assert L == 50 and N % (NW * T * NBUF) == 0
    n_chunks = N // (NW * T)
    x2 = X.reshape(NW, n_chunks, T).astype(jnp.int32)
    ht_t = hash_table.T.reshape(-1)
    imp_t = word_importance.T.reshape(-1)

    mesh = plsc.VectorSubcoreMesh(core_axis_name="c", subcore_axis_name="s",
                                  num_cores=NC, num_subcores=NS)
    run = functools.partial(
        pl.kernel,
        out_type=jax.ShapeDtypeStruct((B, L, WIDTH), jnp.float32),
        mesh=mesh,
        compiler_params=pltpu.CompilerParams(use_tc_tiling_on_sc=False),
        scratch_types=[
            pltpu.VMEM((n_chunks, T), jnp.int32),
            pltpu.VMEM((n_chunks, T), jnp.int32),
            pltpu.VMEM((NBUF, NUM_HASH, T), jnp.int32),
            pltpu.VMEM((NBUF, NUM_HASH, T), jnp.float32),
            pltpu.VMEM((NBUF, NUM_HASH, T, WIDTH), jnp.float32),
            pltpu.VMEM((ROWS, 50, WIDTH), jnp.float32),
            [pltpu.SemaphoreType.DMA] * NBUF,
            [pltpu.SemaphoreType.DMA] * NBUF,
            [pltpu.SemaphoreType.DMA] * NBUF,
        ],
    )(_body)
    return run(x2, ht_t, imp_t, embedding)


# speedup vs baseline: 1.0077x; 1.0077x over previous
import functools

import jax
import jax.numpy as jnp
from jax import lax
from jax.experimental import pallas as pl
from jax.experimental.pallas import tpu as pltpu
from jax.experimental.pallas import tpu_sc as plsc

NUM_WORDS = 1000000
NUM_HASH = 3
WIDTH = 32

NC = 2
NS = 16
NW = NC * NS
LANES = 16
ROWS = 8
T = ROWS * 50
NBUF = 2
PIECES = [(0, T)]


def _body(x_hbm, ht_hbm, imp_hbm, emb_hbm, out_hbm,
          x_v, iidx_v, hrows_v, irows_v, emb_v, out_v,
          sem_h, sem_i, sem_e):
    n_chunks = x_v.shape[0]
    G = T // LANES
    wid = lax.axis_index("s") * NC + lax.axis_index("c")

    pltpu.sync_copy(x_hbm.at[wid], x_v)

    def idx_body(i, _):
        c = i // G
        sl = pl.ds((i % G) * LANES, LANES)
        xi = x_v[c, sl] + 3
        iidx_v[c, sl] = jnp.where(xi >= NUM_WORDS, xi - NUM_WORDS, xi)
        return _
    lax.fori_loop(0, n_chunks * G, idx_body, None)

    def issue_h(c, b):
        for j in range(NUM_HASH):
            seg = ht_hbm.at[pl.ds(j * NUM_WORDS, NUM_WORDS)]
            for o, s in PIECES:
                pltpu.async_copy(seg.at[x_v.at[c, pl.ds(o, s)]],
                                 hrows_v.at[b, j, pl.ds(o, s)], sem_h[b])

    def issue_i(c, b):
        for j in range(NUM_HASH):
            seg = imp_hbm.at[pl.ds(j * NUM_WORDS, NUM_WORDS)]
            for o, s in PIECES:
                pltpu.async_copy(seg.at[iidx_v.at[c, pl.ds(o, s)]],
                                 irows_v.at[b, j, pl.ds(o, s)], sem_i[b])

    def issue_hi(c, b):
        issue_h(c, b)
        issue_i(c, b)

    def wait_h(c, b):
        for j in range(NUM_HASH):
            seg = ht_hbm.at[pl.ds(j * NUM_WORDS, NUM_WORDS)]
            for o, s in PIECES:
                pltpu.make_async_copy(seg.at[x_v.at[c, pl.ds(o, s)]],
                                      hrows_v.at[b, j, pl.ds(o, s)],
                                      sem_h[b]).wait()

    def wait_i(c, b):
        for j in range(NUM_HASH):
            seg = imp_hbm.at[pl.ds(j * NUM_WORDS, NUM_WORDS)]
            for o, s in PIECES:
                pltpu.make_async_copy(seg.at[iidx_v.at[c, pl.ds(o, s)]],
                                      irows_v.at[b, j, pl.ds(o, s)],
                                      sem_i[b]).wait()

    def issue_e(b):
        for j in range(NUM_HASH):
            pltpu.async_copy(emb_hbm.at[hrows_v.at[b, j]], emb_v.at[b, j],
                             sem_e[b])

    def wait_e(b):
        for j in range(NUM_HASH):
            pltpu.make_async_copy(emb_hbm.at[hrows_v.at[b, j]],
                                  emb_v.at[b, j], sem_e[b]).wait()

    def compute(c, b):
        def grp_body(g, _):
            sl = pl.ds(g * LANES, LANES)
            vi = [irows_v[b, j, sl] for j in range(NUM_HASH)]
            for k in range(LANES):
                t = g * LANES + k
                r = t // 50
                l = t - r * 50
                w0, w1, w2 = vi[0][k], vi[1][k], vi[2][k]
                for h in range(0, WIDTH, LANES):
                    acc = (w0 * emb_v[b, 0, t, pl.ds(h, LANES)]
                           + w1 * emb_v[b, 1, t, pl.ds(h, LANES)]
                           + w2 * emb_v[b, 2, t, pl.ds(h, LANES)])
                    out_v[r, l, pl.ds(h, LANES)] = acc
            return _
        lax.fori_loop(0, G, grp_body, None)
        pltpu.sync_copy(out_v,
                        out_hbm.at[pl.ds(wid * (n_chunks * ROWS) + c * ROWS,
                                         ROWS)])

    issue_hi(0, 0)
    wait_h(0, 0)
    issue_e(0)
    issue_hi(1, 1)

    def step(c, b):
        b2 = (b + 1) % NBUF

        @pl.when(c + 1 < n_chunks)
        def _():
            wait_h(c + 1, b2)
            issue_e(b2)
        wait_e(b)
        wait_i(c, b)

        @pl.when(c + 2 < n_chunks)
        def _():
            issue_h(c + 2, b)
        compute(c, b)

        @pl.when(c + 2 < n_chunks)
        def _():
            issue_i(c + 2, b)

    def loop_body(i, _):
        for b in range(NBUF):
            step(i * NBUF + b, b)
        return _
    lax.fori_loop(0, n_chunks // NBUF, loop_body, None)


def kernel(X, hash_table, word_importance, embedding):
    B, L = X.shape
    N = B * L
    assert L == 50 and N % (NW * T * NBUF) == 0
    n_chunks = N // (NW * T)
    x2 = X.reshape(NW, n_chunks, T).astype(jnp.int32)
    ht_t = hash_table.T.reshape(-1)
    imp_t = word_importance.T.reshape(-1)

    mesh = plsc.VectorSubcoreMesh(core_axis_name="c", subcore_axis_name="s",
                                  num_cores=NC, num_subcores=NS)
    run = functools.partial(
        pl.kernel,
        out_type=jax.ShapeDtypeStruct((B, L, WIDTH), jnp.float32),
        mesh=mesh,
        compiler_params=pltpu.CompilerParams(use_tc_tiling_on_sc=False),
        scratch_types=[
            pltpu.VMEM((n_chunks, T), jnp.int32),
            pltpu.VMEM((n_chunks, T), jnp.int32),
            pltpu.VMEM((NBUF, NUM_HASH, T), jnp.int32),
            pltpu.VMEM((NBUF, NUM_HASH, T), jnp.float32),
            pltpu.VMEM((NBUF, NUM_HASH, T, WIDTH), jnp.float32),
            pltpu.VMEM((ROWS, 50, WIDTH), jnp.float32),
            [pltpu.SemaphoreType.DMA] * NBUF,
            [pltpu.SemaphoreType.DMA] * NBUF,
            [pltpu.SemaphoreType.DMA] * NBUF,
        ],
    )(_body)
    return run(x2, ht_t, imp_t, embedding)
